# Initial kernel scaffold; baseline (speedup 1.0000x reference)
import jax
import jax.numpy as jnp
from jax.experimental import pallas as pl


def kernel(x, theta_W, theta_b, phi_W, phi_b):
    raise NotImplementedError("write your pallas kernel here")


# speedup vs baseline: 10.9614x; 10.9614x over previous
import functools

import jax
import jax.numpy as jnp
from jax import lax
from jax.experimental import pallas as pl
from jax.experimental.pallas import tpu as pltpu
from jax.experimental.pallas import tpu_sc as plsc

KNN = 16
EMB = 256
ROW_BLK = 256


def _tc_body(x_rows_ref, x_all_ref, theta_ref, phi_ref, bias_ref,
             idx_ref, t_ref, base_ref):
    b = pl.program_id(0)
    xr = x_rows_ref[0]
    xa = x_all_ref[0]
    n = xa.shape[0]

    g = lax.dot_general(xr, xa, (((1,), (1,)), ((), ())),
                        preferred_element_type=jnp.float32)
    sqr = jnp.sum(xr * xr, axis=1)
    sqa = jnp.sum(xa * xa, axis=1)
    d = sqr[:, None] - 2.0 * g + sqa[None, :]

    col = lax.broadcasted_iota(jnp.int32, (ROW_BLK, n), 1)
    kcol = lax.broadcasted_iota(jnp.int32, (ROW_BLK, KNN), 1)
    idx_acc = jnp.zeros((ROW_BLK, KNN), jnp.int32)
    for k in range(KNN):
        m = jnp.min(d, axis=1, keepdims=True)
        cand = jnp.where(d == m, col, n)
        j = jnp.min(cand, axis=1, keepdims=True)
        idx_acc = jnp.where(kcol == k, j + b * n, idx_acc)
        d = jnp.where(col == j, jnp.float32(jnp.inf), d)
    idx_ref[0] = idx_acc

    t = lax.dot_general(xr, theta_ref[...], (((1,), (1,)), ((), ())),
                        preferred_element_type=jnp.float32)
    p = lax.dot_general(xr, phi_ref[...], (((1,), (1,)), ((), ())),
                        preferred_element_type=jnp.float32)
    t_ref[0] = t
    base_ref[0] = p - t + bias_ref[...]


def _tc_stage(x, theta_W, phi_W, bias):
    B, N, D = x.shape
    grid = (B, N // ROW_BLK)
    return pl.pallas_call(
        _tc_body,
        grid=grid,
        in_specs=[
            pl.BlockSpec((1, ROW_BLK, D), lambda b, r: (b, r, 0)),
            pl.BlockSpec((1, N, D), lambda b, r: (b, 0, 0)),
            pl.BlockSpec((EMB, D), lambda b, r: (0, 0)),
            pl.BlockSpec((EMB, D), lambda b, r: (0, 0)),
            pl.BlockSpec((1, EMB), lambda b, r: (0, 0)),
        ],
        out_specs=[
            pl.BlockSpec((1, ROW_BLK, KNN), lambda b, r: (b, r, 0)),
            pl.BlockSpec((1, ROW_BLK, EMB), lambda b, r: (b, r, 0)),
            pl.BlockSpec((1, ROW_BLK, EMB), lambda b, r: (b, r, 0)),
        ],
        out_shape=[
            jax.ShapeDtypeStruct((B, N, KNN), jnp.int32),
            jax.ShapeDtypeStruct((B, N, EMB), jnp.float32),
            jax.ShapeDtypeStruct((B, N, EMB), jnp.float32),
        ],
    )(x, x, theta_W, phi_W, bias)


def _sc_gather_max(t_flat, base_flat, idx_flat):
    BN = t_flat.shape[0]
    info = plsc.get_sparse_core_info()
    nw = info.num_cores * info.num_subcores
    pts_per_w = BN // nw
    PC = 8
    n_chunks = pts_per_w // PC
    mesh = plsc.VectorSubcoreMesh(core_axis_name="c", subcore_axis_name="s")

    @functools.partial(
        pl.kernel, mesh=mesh,
        out_type=jax.ShapeDtypeStruct((BN, EMB), jnp.float32),
        scratch_types=[
            pltpu.VMEM((PC * KNN,), jnp.int32),
            pltpu.VMEM((PC * KNN, EMB), jnp.float32),
            pltpu.VMEM((PC, EMB), jnp.float32),
            pltpu.SemaphoreType.DMA,
        ],
    )
    def sc_kernel(t_hbm, base_hbm, idx_hbm, out_hbm, idx_v, rows_v, out_v, sem):
        wid = lax.axis_index("s") * info.num_cores + lax.axis_index("c")
        w_base = wid * pts_per_w

        def chunk_body(c, carry):
            p0 = w_base + c * PC
            pltpu.sync_copy(idx_hbm.at[pl.ds(p0 * KNN, PC * KNN)], idx_v)
            pltpu.async_copy(t_hbm.at[idx_v], rows_v, sem).wait()
            pltpu.sync_copy(base_hbm.at[pl.ds(p0, PC)], out_v)

            def col_body(gidx, inner):
                c0 = gidx * 16
                for p in range(PC):
                    acc = rows_v[p * KNN, pl.ds(c0, 16)]
                    for r in range(1, KNN):
                        acc = jnp.maximum(acc, rows_v[p * KNN + r, pl.ds(c0, 16)])
                    out_v[p, pl.ds(c0, 16)] = out_v[p, pl.ds(c0, 16)] + acc
                return inner

            lax.fori_loop(0, EMB // 16, col_body, 0)
            pltpu.sync_copy(out_v, out_hbm.at[pl.ds(p0, PC)])
            return carry

        lax.fori_loop(0, n_chunks, chunk_body, 0)

    return sc_kernel(t_flat, base_flat, idx_flat)


def kernel(x, theta_W, theta_b, phi_W, phi_b):
    B, N, D = x.shape
    bias = (theta_b + phi_b).reshape(1, EMB)
    idx, t, base = _tc_stage(x, theta_W, phi_W, bias)
    out = _sc_gather_max(
        t.reshape(B * N, EMB),
        base.reshape(B * N, EMB),
        idx.reshape(B * N * KNN),
    )
    return out


# speedup vs baseline: 12.3812x; 1.1295x over previous
import functools

import jax
import jax.numpy as jnp
from jax import lax
from jax.experimental import pallas as pl
from jax.experimental.pallas import tpu as pltpu
from jax.experimental.pallas import tpu_sc as plsc

KNN = 16
EMB = 256
ROW_BLK = 256


def _tc_body(x_rows_ref, x_all_ref, theta_ref, phi_ref, bias_ref,
             idx_ref, t_ref, base_ref):
    b = pl.program_id(0)
    xr = x_rows_ref[0]
    xa = x_all_ref[0]
    n = xa.shape[0]

    g = lax.dot_general(xr, xa, (((1,), (1,)), ((), ())),
                        preferred_element_type=jnp.float32)
    sqr = jnp.sum(xr * xr, axis=1)
    sqa = jnp.sum(xa * xa, axis=1)
    d = sqr[:, None] - 2.0 * g + sqa[None, :]

    col = lax.broadcasted_iota(jnp.int32, (ROW_BLK, n), 1)
    kcol = lax.broadcasted_iota(jnp.int32, (ROW_BLK, KNN), 1)
    idx_acc = jnp.zeros((ROW_BLK, KNN), jnp.int32)
    for k in range(KNN):
        m = jnp.min(d, axis=1, keepdims=True)
        cand = jnp.where(d == m, col, n)
        j = jnp.min(cand, axis=1, keepdims=True)
        idx_acc = jnp.where(kcol == k, j + b * n, idx_acc)
        d = jnp.where(col == j, jnp.float32(jnp.inf), d)
    idx_ref[0] = idx_acc

    t = lax.dot_general(xr, theta_ref[...], (((1,), (1,)), ((), ())),
                        preferred_element_type=jnp.float32)
    p = lax.dot_general(xr, phi_ref[...], (((1,), (1,)), ((), ())),
                        preferred_element_type=jnp.float32)
    t_ref[0] = t
    base_ref[0] = p - t + bias_ref[...]


def _tc_stage(x, theta_W, phi_W, bias):
    B, N, D = x.shape
    grid = (B, N // ROW_BLK)
    return pl.pallas_call(
        _tc_body,
        grid=grid,
        in_specs=[
            pl.BlockSpec((1, ROW_BLK, D), lambda b, r: (b, r, 0)),
            pl.BlockSpec((1, N, D), lambda b, r: (b, 0, 0)),
            pl.BlockSpec((EMB, D), lambda b, r: (0, 0)),
            pl.BlockSpec((EMB, D), lambda b, r: (0, 0)),
            pl.BlockSpec((1, EMB), lambda b, r: (0, 0)),
        ],
        out_specs=[
            pl.BlockSpec((1, ROW_BLK, KNN), lambda b, r: (b, r, 0)),
            pl.BlockSpec((1, ROW_BLK, EMB), lambda b, r: (b, r, 0)),
            pl.BlockSpec((1, ROW_BLK, EMB), lambda b, r: (b, r, 0)),
        ],
        out_shape=[
            jax.ShapeDtypeStruct((B, N, KNN), jnp.int32),
            jax.ShapeDtypeStruct((B, N, EMB), jnp.float32),
            jax.ShapeDtypeStruct((B, N, EMB), jnp.float32),
        ],
    )(x, x, theta_W, phi_W, bias)


def _sc_gather_max(t_flat, base_flat, idx_flat):
    BN = t_flat.shape[0]
    info = plsc.get_sparse_core_info()
    nw = info.num_cores * info.num_subcores
    pts_per_w = BN // nw
    PC = 8
    n_chunks = pts_per_w // PC
    mesh = plsc.VectorSubcoreMesh(core_axis_name="c", subcore_axis_name="s")

    @functools.partial(
        pl.kernel, mesh=mesh,
        out_type=jax.ShapeDtypeStruct((BN, EMB), jnp.float32),
        scratch_types=[
            pltpu.VMEM((PC * KNN,), jnp.int32),
            pltpu.VMEM((PC * KNN,), jnp.int32),
            pltpu.VMEM((PC * KNN, EMB), jnp.float32),
            pltpu.VMEM((PC * KNN, EMB), jnp.float32),
            pltpu.VMEM((PC, EMB), jnp.float32),
            pltpu.SemaphoreType.DMA,
            pltpu.SemaphoreType.DMA,
        ],
    )
    def sc_kernel(t_hbm, base_hbm, idx_hbm, out_hbm,
                  idx_v0, idx_v1, rows_v0, rows_v1, out_v, sem0, sem1):
        wid = lax.axis_index("s") * info.num_cores + lax.axis_index("c")
        w_base = wid * pts_per_w

        def start_gather(c, idx_v, rows_v, sem):
            p0 = w_base + c * PC
            pltpu.sync_copy(idx_hbm.at[pl.ds(p0 * KNN, PC * KNN)], idx_v)
            pltpu.async_copy(t_hbm.at[idx_v], rows_v, sem)

        def compute(c, idx_v, rows_v, sem):
            p0 = w_base + c * PC
            pltpu.make_async_copy(t_hbm.at[idx_v], rows_v, sem).wait()
            pltpu.sync_copy(base_hbm.at[pl.ds(p0, PC)], out_v)

            def col_body(gidx, inner):
                c0 = gidx * 16
                for p in range(PC):
                    acc = rows_v[p * KNN, pl.ds(c0, 16)]
                    for r in range(1, KNN):
                        acc = jnp.maximum(acc, rows_v[p * KNN + r, pl.ds(c0, 16)])
                    out_v[p, pl.ds(c0, 16)] = out_v[p, pl.ds(c0, 16)] + acc
                return inner

            lax.fori_loop(0, EMB // 16, col_body, 0)
            pltpu.sync_copy(out_v, out_hbm.at[pl.ds(p0, PC)])

        start_gather(0, idx_v0, rows_v0, sem0)

        def body(g, carry):
            c0 = 2 * g
            start_gather(c0 + 1, idx_v1, rows_v1, sem1)
            compute(c0, idx_v0, rows_v0, sem0)

            @pl.when(g < n_chunks // 2 - 1)
            def _():
                start_gather(c0 + 2, idx_v0, rows_v0, sem0)

            compute(c0 + 1, idx_v1, rows_v1, sem1)
            return carry

        lax.fori_loop(0, n_chunks // 2, body, 0)

    return sc_kernel(t_flat, base_flat, idx_flat)


def kernel(x, theta_W, theta_b, phi_W, phi_b):
    B, N, D = x.shape
    bias = (theta_b + phi_b).reshape(1, EMB)
    idx, t, base = _tc_stage(x, theta_W, phi_W, bias)
    out = _sc_gather_max(
        t.reshape(B * N, EMB),
        base.reshape(B * N, EMB),
        idx.reshape(B * N * KNN),
    )
    return out


# speedup vs baseline: 13.7083x; 1.1072x over previous
import functools

import jax
import jax.numpy as jnp
from jax import lax
from jax.experimental import pallas as pl
from jax.experimental.pallas import tpu as pltpu
from jax.experimental.pallas import tpu_sc as plsc

KNN = 16
EMB = 256
HALF = EMB // 2
ROW_BLK = 256


def _tc_body(x_rows_ref, x_all_ref, theta_ref, phi_ref, bias_ref,
             idx_ref, tw_ref, base_ref):
    b = pl.program_id(0)
    xr = x_rows_ref[0]
    xa = x_all_ref[0]
    n = xa.shape[0]

    g = lax.dot_general(xr, xa, (((1,), (1,)), ((), ())),
                        preferred_element_type=jnp.float32)
    sqr = jnp.sum(xr * xr, axis=1)
    sqa = jnp.sum(xa * xa, axis=1)
    d = sqr[:, None] - 2.0 * g + sqa[None, :]

    col = lax.broadcasted_iota(jnp.int32, (ROW_BLK, n), 1)
    kcol = lax.broadcasted_iota(jnp.int32, (ROW_BLK, KNN), 1)
    idx_acc = jnp.zeros((ROW_BLK, KNN), jnp.int32)
    for k in range(KNN):
        m = jnp.min(d, axis=1, keepdims=True)
        cand = jnp.where(d == m, col, n)
        j = jnp.min(cand, axis=1, keepdims=True)
        idx_acc = jnp.where(kcol == k, j + b * n, idx_acc)
        d = jnp.where(col == j, jnp.float32(jnp.inf), d)
    idx_ref[0] = idx_acc

    t = lax.dot_general(xr, theta_ref[...], (((1,), (1,)), ((), ())),
                        preferred_element_type=jnp.float32)
    p = lax.dot_general(xr, phi_ref[...], (((1,), (1,)), ((), ())),
                        preferred_element_type=jnp.float32)
    base_ref[0] = p - t + bias_ref[...]

    y = t.astype(jnp.bfloat16)
    lo = lax.bitcast_convert_type(lax.slice(y, (0, 0), (ROW_BLK, HALF)),
                                  jnp.uint16).astype(jnp.uint32)
    hi = lax.bitcast_convert_type(lax.slice(y, (0, HALF), (ROW_BLK, EMB)),
                                  jnp.uint16).astype(jnp.uint32)
    tw_ref[0] = lax.bitcast_convert_type(lo | (hi << 16), jnp.int32)


def _tc_stage(x, theta_W, phi_W, bias):
    B, N, D = x.shape
    grid = (B, N // ROW_BLK)
    return pl.pallas_call(
        _tc_body,
        grid=grid,
        in_specs=[
            pl.BlockSpec((1, ROW_BLK, D), lambda b, r: (b, r, 0)),
            pl.BlockSpec((1, N, D), lambda b, r: (b, 0, 0)),
            pl.BlockSpec((EMB, D), lambda b, r: (0, 0)),
            pl.BlockSpec((EMB, D), lambda b, r: (0, 0)),
            pl.BlockSpec((1, EMB), lambda b, r: (0, 0)),
        ],
        out_specs=[
            pl.BlockSpec((1, ROW_BLK, KNN), lambda b, r: (b, r, 0)),
            pl.BlockSpec((1, ROW_BLK, HALF), lambda b, r: (b, r, 0)),
            pl.BlockSpec((1, ROW_BLK, EMB), lambda b, r: (b, r, 0)),
        ],
        out_shape=[
            jax.ShapeDtypeStruct((B, N, KNN), jnp.int32),
            jax.ShapeDtypeStruct((B, N, HALF), jnp.int32),
            jax.ShapeDtypeStruct((B, N, EMB), jnp.float32),
        ],
    )(x, x, theta_W, phi_W, bias)


def _sc_gather_max(tw_flat, idx_flat):
    BN = tw_flat.shape[0]
    info = plsc.get_sparse_core_info()
    nw = info.num_cores * info.num_subcores
    pts_per_w = BN // nw
    PC = 8
    n_chunks = pts_per_w // PC
    mesh = plsc.VectorSubcoreMesh(core_axis_name="c", subcore_axis_name="s")

    @functools.partial(
        pl.kernel, mesh=mesh,
        out_type=jax.ShapeDtypeStruct((BN, HALF), jnp.int32),
        scratch_types=[
            pltpu.VMEM((PC * KNN,), jnp.int32),
            pltpu.VMEM((PC * KNN,), jnp.int32),
            pltpu.VMEM((PC * KNN, HALF), jnp.int32),
            pltpu.VMEM((PC * KNN, HALF), jnp.int32),
            pltpu.VMEM((PC, HALF), jnp.int32),
            pltpu.SemaphoreType.DMA,
            pltpu.SemaphoreType.DMA,
        ],
    )
    def sc_kernel(tw_hbm, idx_hbm, out_hbm,
                  idx_v0, idx_v1, rows_v0, rows_v1, out_v, sem0, sem1):
        wid = lax.axis_index("s") * info.num_cores + lax.axis_index("c")
        w_base = wid * pts_per_w

        def start_gather(c, idx_v, rows_v, sem):
            p0 = w_base + c * PC
            pltpu.sync_copy(idx_hbm.at[pl.ds(p0 * KNN, PC * KNN)], idx_v)
            pltpu.async_copy(tw_hbm.at[idx_v], rows_v, sem)

        def compute(c, idx_v, rows_v, sem):
            p0 = w_base + c * PC
            pltpu.make_async_copy(tw_hbm.at[idx_v], rows_v, sem).wait()

            himask = jnp.full((16,), -65536, jnp.int32)

            def unpack2(w):
                lo = lax.bitcast_convert_type(w << 16, jnp.float32)
                hi = lax.bitcast_convert_type(w & himask, jnp.float32)
                return lo, hi

            def col_body(gidx, inner):
                c0 = gidx * 16
                for p in range(PC):
                    alo, ahi = unpack2(rows_v[p * KNN, pl.ds(c0, 16)])
                    for r in range(1, KNN):
                        blo, bhi = unpack2(rows_v[p * KNN + r, pl.ds(c0, 16)])
                        alo = jnp.maximum(alo, blo)
                        ahi = jnp.maximum(ahi, bhi)
                    wlo = lax.shift_right_logical(
                        lax.bitcast_convert_type(alo, jnp.int32), 16)
                    out_v[p, pl.ds(c0, 16)] = (
                        wlo | lax.bitcast_convert_type(ahi, jnp.int32))
                return inner

            lax.fori_loop(0, HALF // 16, col_body, 0)
            pltpu.sync_copy(out_v, out_hbm.at[pl.ds(p0, PC)])

        start_gather(0, idx_v0, rows_v0, sem0)

        def body(g, carry):
            c0 = 2 * g
            start_gather(c0 + 1, idx_v1, rows_v1, sem1)
            compute(c0, idx_v0, rows_v0, sem0)

            @pl.when(g < n_chunks // 2 - 1)
            def _():
                start_gather(c0 + 2, idx_v0, rows_v0, sem0)

            compute(c0 + 1, idx_v1, rows_v1, sem1)
            return carry

        lax.fori_loop(0, n_chunks // 2, body, 0)

    return sc_kernel(tw_flat, idx_flat)


def _unpack_body(base_ref, w_ref, out_ref):
    w = w_ref[...]
    lo = lax.bitcast_convert_type(w << 16, jnp.float32)
    hi = lax.bitcast_convert_type(w & jnp.int32(-65536), jnp.float32)
    out_ref[...] = base_ref[...] + jnp.concatenate([lo, hi], axis=1)


def _unpack_stage(base_flat, maxw):
    BN = base_flat.shape[0]
    blk = 2048
    return pl.pallas_call(
        _unpack_body,
        grid=(BN // blk,),
        in_specs=[
            pl.BlockSpec((blk, EMB), lambda i: (i, 0)),
            pl.BlockSpec((blk, HALF), lambda i: (i, 0)),
        ],
        out_specs=pl.BlockSpec((blk, EMB), lambda i: (i, 0)),
        out_shape=jax.ShapeDtypeStruct((BN, EMB), jnp.float32),
    )(base_flat, maxw)


def kernel(x, theta_W, theta_b, phi_W, phi_b):
    B, N, D = x.shape
    bias = (theta_b + phi_b).reshape(1, EMB)
    idx, tw, base = _tc_stage(x, theta_W, phi_W, bias)
    maxw = _sc_gather_max(tw.reshape(B * N, HALF), idx.reshape(B * N * KNN))
    return _unpack_stage(base.reshape(B * N, EMB), maxw)


# speedup vs baseline: 17.7387x; 1.2940x over previous
import functools

import jax
import jax.numpy as jnp
from jax import lax
from jax.experimental import pallas as pl
from jax.experimental.pallas import tpu as pltpu
from jax.experimental.pallas import tpu_sc as plsc

KNN = 16
EMB = 256
HALF = EMB // 2
ROW_BLK = 256


def _tc_body(x_rows_ref, x_all_ref, theta_ref, phi_ref, bias_ref,
             idx_ref, tw_ref, base_ref):
    xr = x_rows_ref[...]
    xa = x_all_ref[...]
    n = xa.shape[0]

    g = lax.dot_general(xr, xa, (((1,), (1,)), ((), ())),
                        preferred_element_type=jnp.float32)
    sqr = jnp.sum(xr * xr, axis=1)
    sqa = jnp.sum(xa * xa, axis=1)
    d = sqr[:, None] - 2.0 * g + sqa[None, :]

    colf = lax.broadcasted_iota(jnp.int32, (ROW_BLK, n), 1).astype(jnp.float32)
    kcol = lax.broadcasted_iota(jnp.int32, (ROW_BLK, KNN), 1)
    idx_acc = jnp.zeros((ROW_BLK, KNN), jnp.int32)
    for k in range(KNN):
        m = jnp.min(d, axis=1, keepdims=True)
        candf = jnp.where(d == m, colf, jnp.float32(n))
        jf = jnp.min(candf, axis=1, keepdims=True)
        idx_acc = jnp.where(kcol == k, jf.astype(jnp.int32), idx_acc)
        d = jnp.where(colf == jf, jnp.float32(jnp.inf), d)
    idx_ref[...] = idx_acc

    t = lax.dot_general(xr, theta_ref[...], (((1,), (1,)), ((), ())),
                        preferred_element_type=jnp.float32)
    p = lax.dot_general(xr, phi_ref[...], (((1,), (1,)), ((), ())),
                        preferred_element_type=jnp.float32)
    base_ref[...] = p - t + bias_ref[...]

    y = t.astype(jnp.bfloat16)
    lo = lax.bitcast_convert_type(lax.slice(y, (0, 0), (ROW_BLK, HALF)),
                                  jnp.uint16).astype(jnp.uint32)
    hi = lax.bitcast_convert_type(lax.slice(y, (0, HALF), (ROW_BLK, EMB)),
                                  jnp.uint16).astype(jnp.uint32)
    tw_ref[...] = lax.bitcast_convert_type(lo | (hi << 16), jnp.int32)


def _tc_stage(xb, theta_W, phi_W, bias):
    N, D = xb.shape
    return pl.pallas_call(
        _tc_body,
        grid=(N // ROW_BLK,),
        in_specs=[
            pl.BlockSpec((ROW_BLK, D), lambda r: (r, 0)),
            pl.BlockSpec((N, D), lambda r: (0, 0)),
            pl.BlockSpec((EMB, D), lambda r: (0, 0)),
            pl.BlockSpec((EMB, D), lambda r: (0, 0)),
            pl.BlockSpec((1, EMB), lambda r: (0, 0)),
        ],
        out_specs=[
            pl.BlockSpec((ROW_BLK, KNN), lambda r: (r, 0)),
            pl.BlockSpec((ROW_BLK, HALF), lambda r: (r, 0)),
            pl.BlockSpec((ROW_BLK, EMB), lambda r: (r, 0)),
        ],
        out_shape=[
            jax.ShapeDtypeStruct((N, KNN), jnp.int32),
            jax.ShapeDtypeStruct((N, HALF), jnp.int32),
            jax.ShapeDtypeStruct((N, EMB), jnp.float32),
        ],
    )(xb, xb, theta_W, phi_W, bias)


def _sc_gather_max(tw_flat, idx_flat):
    BN = tw_flat.shape[0]
    info = plsc.get_sparse_core_info()
    nw = info.num_cores * info.num_subcores
    pts_per_w = BN // nw
    PC = 8
    n_chunks = pts_per_w // PC
    mesh = plsc.VectorSubcoreMesh(core_axis_name="c", subcore_axis_name="s")

    @functools.partial(
        pl.kernel, mesh=mesh,
        out_type=jax.ShapeDtypeStruct((BN, HALF), jnp.int32),
        scratch_types=[
            pltpu.VMEM((PC * KNN,), jnp.int32),
            pltpu.VMEM((PC * KNN,), jnp.int32),
            pltpu.VMEM((PC * KNN, HALF), jnp.int32),
            pltpu.VMEM((PC * KNN, HALF), jnp.int32),
            pltpu.VMEM((PC, HALF), jnp.int32),
            pltpu.SemaphoreType.DMA,
            pltpu.SemaphoreType.DMA,
        ],
    )
    def sc_kernel(tw_hbm, idx_hbm, out_hbm,
                  idx_v0, idx_v1, rows_v0, rows_v1, out_v, sem0, sem1):
        wid = lax.axis_index("s") * info.num_cores + lax.axis_index("c")
        w_base = wid * pts_per_w

        def start_gather(c, idx_v, rows_v, sem):
            p0 = w_base + c * PC
            pltpu.sync_copy(idx_hbm.at[pl.ds(p0 * KNN, PC * KNN)], idx_v)
            pltpu.async_copy(tw_hbm.at[idx_v], rows_v, sem)

        def compute(c, idx_v, rows_v, sem):
            p0 = w_base + c * PC
            pltpu.make_async_copy(tw_hbm.at[idx_v], rows_v, sem).wait()

            himask = jnp.full((16,), -65536, jnp.int32)

            def unpack2(w):
                lo = lax.bitcast_convert_type(w << 16, jnp.float32)
                hi = lax.bitcast_convert_type(w & himask, jnp.float32)
                return lo, hi

            def col_body(gidx, inner):
                c0 = gidx * 16
                for p in range(PC):
                    alo, ahi = unpack2(rows_v[p * KNN, pl.ds(c0, 16)])
                    for r in range(1, KNN):
                        blo, bhi = unpack2(rows_v[p * KNN + r, pl.ds(c0, 16)])
                        alo = jnp.maximum(alo, blo)
                        ahi = jnp.maximum(ahi, bhi)
                    wlo = lax.shift_right_logical(
                        lax.bitcast_convert_type(alo, jnp.int32), 16)
                    out_v[p, pl.ds(c0, 16)] = (
                        wlo | lax.bitcast_convert_type(ahi, jnp.int32))
                return inner

            lax.fori_loop(0, HALF // 16, col_body, 0)
            pltpu.sync_copy(out_v, out_hbm.at[pl.ds(p0, PC)])

        start_gather(0, idx_v0, rows_v0, sem0)

        def body(g, carry):
            c0 = 2 * g
            start_gather(c0 + 1, idx_v1, rows_v1, sem1)
            compute(c0, idx_v0, rows_v0, sem0)

            @pl.when(g < n_chunks // 2 - 1)
            def _():
                start_gather(c0 + 2, idx_v0, rows_v0, sem0)

            compute(c0 + 1, idx_v1, rows_v1, sem1)
            return carry

        lax.fori_loop(0, n_chunks // 2, body, 0)

    return sc_kernel(tw_flat, idx_flat)


def _unpack_body(base_ref, w_ref, out_ref):
    w = w_ref[...]
    lo = lax.bitcast_convert_type(w << 16, jnp.float32)
    hi = lax.bitcast_convert_type(w & jnp.int32(-65536), jnp.float32)
    out_ref[...] = base_ref[...] + jnp.concatenate([lo, hi], axis=1)


def _unpack_stage(base_flat, maxw):
    BN = base_flat.shape[0]
    blk = 2048
    return pl.pallas_call(
        _unpack_body,
        grid=(BN // blk,),
        in_specs=[
            pl.BlockSpec((blk, EMB), lambda i: (i, 0)),
            pl.BlockSpec((blk, HALF), lambda i: (i, 0)),
        ],
        out_specs=pl.BlockSpec((blk, EMB), lambda i: (i, 0)),
        out_shape=jax.ShapeDtypeStruct((BN, EMB), jnp.float32),
    )(base_flat, maxw)


def kernel(x, theta_W, theta_b, phi_W, phi_b):
    B, N, D = x.shape
    bias = (theta_b + phi_b).reshape(1, EMB)
    outs = []
    for b in range(B):
        idx_b, tw_b, base_b = _tc_stage(x[b], theta_W, phi_W, bias)
        maxw_b = _sc_gather_max(tw_b, idx_b.reshape(N * KNN))
        outs.append(_unpack_stage(base_b, maxw_b))
    return jnp.concatenate(outs, axis=0)
